# Initial kernel scaffold; baseline (speedup 1.0000x reference)
import functools

import jax
import jax.numpy as jnp
from jax import lax
from jax.experimental import pallas as pl
from jax.experimental.pallas import tpu as pltpu
from jax.experimental.pallas import tpu_sc as plsc

_BATCH = 4096
_HIST = 200
_DIM = 32
_B = _BATCH * _HIST
_NC = 2
_NS = 16
_NW = _NC * _NS
_B_PER_W = _B // _NW
_GROUP = 128
_G = 8
_CHUNK = _GROUP * _G
_N_CHUNKS = _B_PER_W // _CHUNK


def _build():
    mesh = plsc.VectorSubcoreMesh(core_axis_name="c", subcore_axis_name="s")

    @functools.partial(
        pl.kernel,
        out_type=jax.ShapeDtypeStruct((_B, _DIM), jnp.float32),
        mesh=mesh,
        scratch_types=[
            pltpu.VMEM((_G, _GROUP), jnp.int32),
            pltpu.VMEM((_CHUNK, _DIM), jnp.float32),
            pltpu.SemaphoreType.DMA,
        ],
    )
    def gather_kernel(idx_hbm, table_hbm, out_hbm, idx_v, rows_v, sem):
        wid = lax.axis_index("s") * _NC + lax.axis_index("c")
        row_base = wid * _B_PER_W
        grp_base = wid * (_B_PER_W // _GROUP)

        def chunk_body(ci, carry):
            row0 = row_base + ci * _CHUNK
            grp0 = grp_base + ci * _G
            pltpu.sync_copy(idx_hbm.at[pl.ds(grp0, _G)], idx_v)
            copies = []
            for j in range(_G):
                copies.append(
                    pltpu.async_copy(
                        table_hbm.at[idx_v.at[j]],
                        rows_v.at[pl.ds(j * _GROUP, _GROUP)],
                        sem,
                    )
                )
            for c in copies:
                c.wait()
            pltpu.sync_copy(rows_v, out_hbm.at[pl.ds(row0, _CHUNK)])
            return carry

        lax.fori_loop(0, _N_CHUNKS, chunk_body, 0)

    return gather_kernel


_gather = _build()


def kernel(x, table):
    idx = x.astype(jnp.int32).reshape(_B // _GROUP, _GROUP)
    out = _gather(idx, table)
    return out.reshape(_BATCH, _HIST, _DIM)


# speedup vs baseline: 1.4589x; 1.4589x over previous
import functools

import jax
import jax.numpy as jnp
from jax import lax
from jax.experimental import pallas as pl
from jax.experimental.pallas import tpu as pltpu
from jax.experimental.pallas import tpu_sc as plsc

_BATCH = 4096
_HIST = 200
_DIM = 32
_B = _BATCH * _HIST
_NC = 2
_NS = 16
_NW = _NC * _NS
_B_PER_W = _B // _NW
_GROUP = 128
_G = 8
_CHUNK = _GROUP * _G
_N_CHUNKS = _B_PER_W // _CHUNK


def _build():
    mesh = plsc.VectorSubcoreMesh(core_axis_name="c", subcore_axis_name="s")

    @functools.partial(
        pl.kernel,
        out_type=jax.ShapeDtypeStruct((_B, _DIM), jnp.float32),
        mesh=mesh,
        compiler_params=pltpu.CompilerParams(use_tc_tiling_on_sc=False),
        scratch_types=[
            pltpu.VMEM((_G, _GROUP), jnp.int32),
            pltpu.VMEM((_CHUNK, _DIM), jnp.float32),
            pltpu.SemaphoreType.DMA,
        ],
    )
    def gather_kernel(idx_hbm, table_hbm, out_hbm, idx_v, rows_v, sem):
        wid = lax.axis_index("s") * _NC + lax.axis_index("c")
        row_base = wid * _B_PER_W
        grp_base = wid * (_B_PER_W // _GROUP)

        def chunk_body(ci, carry):
            row0 = row_base + ci * _CHUNK
            grp0 = grp_base + ci * _G
            pltpu.sync_copy(idx_hbm.at[pl.ds(grp0, _G)], idx_v)
            copies = []
            for j in range(_G):
                copies.append(
                    pltpu.async_copy(
                        table_hbm.at[idx_v.at[j]],
                        rows_v.at[pl.ds(j * _GROUP, _GROUP)],
                        sem,
                    )
                )
            for c in copies:
                c.wait()
            pltpu.sync_copy(rows_v, out_hbm.at[pl.ds(row0, _CHUNK)])
            return carry

        lax.fori_loop(0, _N_CHUNKS, chunk_body, 0)

    return gather_kernel


_gather = _build()


def kernel(x, table):
    idx = x.astype(jnp.int32).reshape(_B // _GROUP, _GROUP)
    out = _gather(idx, table)
    return out.reshape(_BATCH, _HIST, _DIM)


# speedup vs baseline: 1.4934x; 1.0236x over previous
import functools

import jax
import jax.numpy as jnp
from jax import lax
from jax.experimental import pallas as pl
from jax.experimental.pallas import tpu as pltpu
from jax.experimental.pallas import tpu_sc as plsc

_BATCH = 4096
_HIST = 200
_DIM = 32
_B = _BATCH * _HIST
_NC = 2
_NS = 16
_NW = _NC * _NS
_B_PER_W = _B // _NW
_CHUNK = 1280
_N_CHUNKS = _B_PER_W // _CHUNK
_PAIRS = _N_CHUNKS // 2


def _build():
    mesh = plsc.VectorSubcoreMesh(core_axis_name="c", subcore_axis_name="s")

    @functools.partial(
        pl.kernel,
        out_type=jax.ShapeDtypeStruct((_B, _DIM), jnp.float32),
        mesh=mesh,
        compiler_params=pltpu.CompilerParams(use_tc_tiling_on_sc=False),
        scratch_types=[
            pltpu.VMEM((_B_PER_W,), jnp.int32),
            pltpu.VMEM((_CHUNK, _DIM), jnp.float32),
            pltpu.VMEM((_CHUNK, _DIM), jnp.float32),
            pltpu.SemaphoreType.DMA,
            pltpu.SemaphoreType.DMA,
            pltpu.SemaphoreType.DMA,
            pltpu.SemaphoreType.DMA,
        ],
    )
    def gather_kernel(idx_hbm, table_hbm, out_hbm, idx_v, buf0, buf1,
                      sem_g0, sem_g1, sem_w0, sem_w1):
        wid = lax.axis_index("s") * _NC + lax.axis_index("c")
        row_base = wid * _B_PER_W
        bufs = (buf0, buf1)
        sems_g = (sem_g0, sem_g1)
        sems_w = (sem_w0, sem_w1)

        pltpu.sync_copy(idx_hbm.at[pl.ds(row_base, _B_PER_W)], idx_v)

        def fire_gather(chunk, b):
            r0 = chunk * _CHUNK
            return pltpu.async_copy(
                table_hbm.at[idx_v.at[pl.ds(r0, _CHUNK)]], bufs[b], sems_g[b])

        def fire_wb(chunk, b):
            r0 = row_base + chunk * _CHUNK
            return pltpu.async_copy(bufs[b], out_hbm.at[pl.ds(r0, _CHUNK)],
                                    sems_w[b])

        def wait_wb(b):
            pltpu.make_async_copy(out_hbm.at[pl.ds(0, _CHUNK)], bufs[b],
                                  sems_w[b]).wait()

        def pair_body(p, carry):
            c0 = 2 * p
            c1 = c0 + 1

            @pl.when(p > 0)
            def _():
                wait_wb(0)
                wait_wb(1)

            g0 = fire_gather(c0, 0)
            g1 = fire_gather(c1, 1)
            g0.wait()
            fire_wb(c0, 0)
            g1.wait()
            fire_wb(c1, 1)
            return carry

        lax.fori_loop(0, _PAIRS, pair_body, 0)
        wait_wb(0)
        wait_wb(1)

    return gather_kernel


_gather = _build()


def kernel(x, table):
    idx = x.astype(jnp.int32).reshape(_B)
    out = _gather(idx, table)
    return out.reshape(_BATCH, _HIST, _DIM)
